# baseline (speedup 1.0000x reference)
import functools

import jax
import jax.numpy as jnp
import numpy as np
from jax import lax
from jax.experimental import pallas as pl
from jax.experimental.pallas import tpu as pltpu
from jax.experimental.pallas import tpu_sc as plsc

_B = 64
_N = 100000
_CH = 128
_NCH = (_N + _CH - 1) // _CH
_PIECE = 16384
_NP = 7
_BUF = _NP * _PIECE
_NG = _BUF // (16 * _CH)
_NSEL = 12
_MAIN = 99840
_NEG = np.float32(-np.inf)


def _splat(x):
    return jnp.full((16,), x, jnp.int32)


def _piece_max(buf_ref, cm_ref, p):
    base = p * _PIECE

    def body(t, accs):
        off = base + t * 128
        return tuple(
            jnp.maximum(a, buf_ref[pl.ds(off + 16 * v, 16)])
            for v, a in enumerate(accs))

    accs = lax.fori_loop(
        0, 128, body,
        tuple(jnp.full((16,), _NEG, jnp.float32) for _ in range(8)))
    for v, a in enumerate(accs):
        cm_ref[pl.ds(p * 128 + 16 * v, 16)] = a


def _process_row(prob_hbm, tail_hbm, ri, buf_ref, cm_ref, ids_ref, lab_vmem,
                 sem_a, sem_b):
    iota = lax.iota(jnp.int32, 16)
    sems = [sem_a, sem_b]
    sizes = [_PIECE] * (_NP - 1) + [_MAIN - (_NP - 1) * _PIECE]

    def fire(p, sem):
        return pltpu.async_copy(
            prob_hbm.at[ri, pl.ds(p * _PIECE, sizes[p])],
            buf_ref.at[pl.ds(p * _PIECE, sizes[p])], sem)

    descs = [None] * _NP
    descs[0] = fire(0, sems[0])
    descs[1] = fire(1, sems[1])
    pltpu.sync_copy(tail_hbm.at[ri], buf_ref.at[pl.ds(_MAIN, 256)])
    for p in range(_NP):
        descs[p].wait()
        if p + 2 < _NP:
            descs[p + 2] = fire(p + 2, sems[p % 2])
        _piece_max(buf_ref, cm_ref, p)

    lane0 = iota == 0
    lab_s = jnp.max(plsc.load_gather(lab_vmem, [_splat(ri)]))
    idg = (lab_s // _PIECE) * 128 + (lab_s % 128)
    plsc.store_scatter(cm_ref, [_splat(idg)], jnp.full((16,), _NEG), mask=lane0)
    plsc.store_scatter(ids_ref, [_splat(_NSEL - 1)], _splat(idg), mask=lane0)

    def sel_body(t, _):
        mv = jnp.full((16,), _NEG, jnp.float32)

        def scan1(v, mv):
            return jnp.maximum(mv, cm_ref[pl.ds(v * 16, 16)])

        mv = lax.fori_loop(0, _NG, scan1, mv)
        mg = jnp.max(mv)

        def scan2(v, pv):
            cv = cm_ref[pl.ds(v * 16, 16)]
            return jnp.minimum(pv, jnp.where(cv == mg, iota + v * 16, _NCH + 999))

        pv = lax.fori_loop(0, _NG, scan2, jnp.full((16,), _NCH + 999, jnp.int32))
        pos = jnp.min(pv)
        plsc.store_scatter(cm_ref, [_splat(pos)], jnp.full((16,), _NEG), mask=lane0)
        plsc.store_scatter(ids_ref, [_splat(t)], _splat(pos), mask=lane0)
        return 0

    lax.fori_loop(0, _NSEL - 1, sel_body, 0)

    st = tuple(jnp.full((16,), _NEG, jnp.float32) for _ in range(11))

    def ins_body(t, st):
        k = t // 8
        j = t - k * 8
        idk = jnp.max(plsc.load_gather(ids_ref, [_splat(k)]))
        base = (idk // 128) * _PIECE + (idk % 128) + 2048 * j
        x = plsc.load_gather(buf_ref, [_splat(base) + iota * 128])
        out = []
        for s in st:
            hi = jnp.maximum(s, x)
            x = jnp.minimum(s, x)
            out.append(hi)
        return tuple(out)

    st = lax.fori_loop(0, _NSEL * 8, ins_body, st)

    def mrg_body(t, carry):
        st, sum11, v11 = carry
        mg = jnp.float32(-jnp.inf)
        for s in st:
            mg = jnp.maximum(mg, jnp.max(s))
        sum11 = sum11 + mg
        pv = jnp.full((16,), 9999, jnp.int32)
        for u, s in enumerate(st):
            pv = jnp.minimum(pv, jnp.where(s == mg, iota + u * 16, 9999))
        pos = jnp.min(pv)
        st = tuple(jnp.where(iota == pos - u * 16, _NEG, s)
                   for u, s in enumerate(st))
        return st, sum11, mg

    _, sum11, v11 = lax.fori_loop(
        0, 11, mrg_body, (st, jnp.float32(0.0), jnp.float32(0.0)))

    c = jnp.max(plsc.load_gather(buf_ref, [_splat(lab_s)]))
    sub = jnp.where(c >= v11, c, v11)
    return (sum11 - sub) * jnp.float32(0.1) - c


def _sc_loss(probas, tail, labels):
    mesh = plsc.VectorSubcoreMesh(core_axis_name="c", subcore_axis_name="s")

    @functools.partial(
        pl.kernel,
        out_type=jax.ShapeDtypeStruct((32, 16), jnp.float32),
        mesh=mesh,
        compiler_params=pltpu.CompilerParams(use_tc_tiling_on_sc=True,
                                             needs_layout_passes=False),
        scratch_types=[
            pltpu.VMEM((_BUF,), jnp.float32),
            pltpu.VMEM((16 * _NG,), jnp.float32),
            pltpu.VMEM((16,), jnp.int32),
            pltpu.VMEM((64,), jnp.int32),
            pltpu.VMEM((16,), jnp.float32),
            pltpu.SemaphoreType.DMA,
            pltpu.SemaphoreType.DMA,
        ],
    )
    def body(prob_hbm, tail_hbm, lab_hbm, out_hbm, buf_ref, cm_ref, ids_ref,
             lab_vmem, stage_ref, sem_a, sem_b):
        wid = lax.axis_index("s") * 2 + lax.axis_index("c")
        pltpu.sync_copy(lab_hbm, lab_vmem)

        def pre(t, _):
            buf_ref[pl.ds(_MAIN + 256 + 16 * t, 16)] = jnp.full(
                (16,), _NEG, jnp.float32)
            return 0

        lax.fori_loop(0, (_BUF - _MAIN - 256) // 16, pre, 0)

        def round_body(r, loss):
            return loss + _process_row(prob_hbm, tail_hbm, 32 * r + wid,
                                       buf_ref, cm_ref, ids_ref, lab_vmem,
                                       sem_a, sem_b)

        loss = lax.fori_loop(0, 2, round_body, jnp.float32(0.0))

        stage_ref[...] = jnp.full((16,), loss, jnp.float32)
        pltpu.sync_copy(stage_ref, out_hbm.at[wid])

    return body(probas, tail, labels)


def _reduce_kernel(part_ref, out_ref):
    out_ref[0, 0] = jnp.sum(part_ref[:, 0:1]) * (1.0 / _B)


@jax.jit
def kernel(probas, labels):
    tail = jnp.pad(probas[:, _MAIN:], ((0, 0), (0, 96)),
                   constant_values=-np.inf)
    partials = _sc_loss(probas, tail, labels.astype(jnp.int32))
    return jnp.sum(partials[:, 0]) * (1.0 / _B)


# speedup vs baseline: 1.1861x; 1.1861x over previous
import functools

import jax
import jax.numpy as jnp
import numpy as np
from jax import lax
from jax.experimental import pallas as pl
from jax.experimental.pallas import tpu as pltpu
from jax.experimental.pallas import tpu_sc as plsc

_B = 64
_N = 100000
_CH = 128
_NCH = (_N + _CH - 1) // _CH
_PIECE = 16384
_NP = 7
_BUF = _NP * _PIECE
_NG = _BUF // (16 * _CH)
_NSEL = 12
_MAIN = 99840
_NEG = np.float32(-np.inf)


def _splat(x):
    return jnp.full((16,), x, jnp.int32)


def _piece_max(buf_ref, cm_ref, p):
    base = p * _PIECE

    def body(t, accs):
        off = base + t * 128
        return tuple(
            jnp.maximum(a, buf_ref[pl.ds(off + 16 * v, 16)])
            for v, a in enumerate(accs))

    accs = lax.fori_loop(
        0, 128, body,
        tuple(jnp.full((16,), _NEG, jnp.float32) for _ in range(8)))
    for v, a in enumerate(accs):
        cm_ref[pl.ds(p * 128 + 16 * v, 16)] = a


def _process_row(prob_hbm, tail_hbm, ri, buf_ref, cm_ref, ids_ref, lab_vmem,
                 sem_a, sem_b):
    iota = lax.iota(jnp.int32, 16)
    sems = [sem_a, sem_b]
    sizes = [_PIECE] * (_NP - 1) + [_MAIN - (_NP - 1) * _PIECE]

    def fire(p, sem):
        return pltpu.async_copy(
            prob_hbm.at[ri, pl.ds(p * _PIECE, sizes[p])],
            buf_ref.at[pl.ds(p * _PIECE, sizes[p])], sem)

    descs = [None] * _NP
    descs[0] = fire(0, sems[0])
    descs[1] = fire(1, sems[1])
    pltpu.sync_copy(tail_hbm.at[ri], buf_ref.at[pl.ds(_MAIN, 256)])
    for p in range(_NP):
        descs[p].wait()
        if p + 2 < _NP:
            descs[p + 2] = fire(p + 2, sems[p % 2])
        _piece_max(buf_ref, cm_ref, p)

    lane0 = iota == 0
    lab_s = jnp.max(plsc.load_gather(lab_vmem, [_splat(ri)]))
    idg = (lab_s // _PIECE) * 128 + (lab_s % 128)
    plsc.store_scatter(cm_ref, [_splat(idg)], jnp.full((16,), _NEG), mask=lane0)
    plsc.store_scatter(ids_ref, [_splat(_NSEL - 1)], _splat(idg), mask=lane0)

    def sel_body(t, _):
        def scan1(v, mvs):
            return tuple(
                jnp.maximum(m, cm_ref[pl.ds((v * 8 + u) * 16, 16)])
                for u, m in enumerate(mvs))

        mvs = lax.fori_loop(
            0, _NG // 8, scan1,
            tuple(jnp.full((16,), _NEG, jnp.float32) for _ in range(8)))
        mv = mvs[0]
        for m in mvs[1:]:
            mv = jnp.maximum(mv, m)
        mg = jnp.max(mv)

        def scan2(v, pvs):
            out = []
            for u, pv in enumerate(pvs):
                cv = cm_ref[pl.ds((v * 8 + u) * 16, 16)]
                out.append(jnp.minimum(
                    pv, jnp.where(cv == mg, iota + (v * 8 + u) * 16, 9999)))
            return tuple(out)

        pvs = lax.fori_loop(
            0, _NG // 8, scan2,
            tuple(jnp.full((16,), 9999, jnp.int32) for _ in range(8)))
        pv = pvs[0]
        for p2 in pvs[1:]:
            pv = jnp.minimum(pv, p2)
        pos = jnp.min(pv)
        plsc.store_scatter(cm_ref, [_splat(pos)], jnp.full((16,), _NEG), mask=lane0)
        plsc.store_scatter(ids_ref, [_splat(t)], _splat(pos), mask=lane0)
        return 0

    lax.fori_loop(0, _NSEL - 1, sel_body, 0)

    st = tuple(jnp.full((16,), _NEG, jnp.float32) for _ in range(11))

    def ins_body(t, st):
        k = t // 8
        j = t - k * 8
        idk = jnp.max(plsc.load_gather(ids_ref, [_splat(k)]))
        base = (idk // 128) * _PIECE + (idk % 128) + 2048 * j
        x = plsc.load_gather(buf_ref, [_splat(base) + iota * 128])
        out = []
        for s in st:
            hi = jnp.maximum(s, x)
            x = jnp.minimum(s, x)
            out.append(hi)
        return tuple(out)

    st = lax.fori_loop(0, _NSEL * 8, ins_body, st)

    def mrg_body(t, carry):
        st, sum11, v11 = carry
        mg = jnp.float32(-jnp.inf)
        for s in st:
            mg = jnp.maximum(mg, jnp.max(s))
        sum11 = sum11 + mg
        pv = jnp.full((16,), 9999, jnp.int32)
        for u, s in enumerate(st):
            pv = jnp.minimum(pv, jnp.where(s == mg, iota + u * 16, 9999))
        pos = jnp.min(pv)
        st = tuple(jnp.where(iota == pos - u * 16, _NEG, s)
                   for u, s in enumerate(st))
        return st, sum11, mg

    _, sum11, v11 = lax.fori_loop(
        0, 11, mrg_body, (st, jnp.float32(0.0), jnp.float32(0.0)))

    c = jnp.max(plsc.load_gather(buf_ref, [_splat(lab_s)]))
    sub = jnp.where(c >= v11, c, v11)
    return (sum11 - sub) * jnp.float32(0.1) - c


def _sc_loss(probas, tail, labels):
    mesh = plsc.VectorSubcoreMesh(core_axis_name="c", subcore_axis_name="s")

    @functools.partial(
        pl.kernel,
        out_type=jax.ShapeDtypeStruct((32, 16), jnp.float32),
        mesh=mesh,
        compiler_params=pltpu.CompilerParams(use_tc_tiling_on_sc=True,
                                             needs_layout_passes=False),
        scratch_types=[
            pltpu.VMEM((_BUF,), jnp.float32),
            pltpu.VMEM((16 * _NG,), jnp.float32),
            pltpu.VMEM((16,), jnp.int32),
            pltpu.VMEM((64,), jnp.int32),
            pltpu.VMEM((16,), jnp.float32),
            pltpu.SemaphoreType.DMA,
            pltpu.SemaphoreType.DMA,
        ],
    )
    def body(prob_hbm, tail_hbm, lab_hbm, out_hbm, buf_ref, cm_ref, ids_ref,
             lab_vmem, stage_ref, sem_a, sem_b):
        wid = lax.axis_index("s") * 2 + lax.axis_index("c")
        pltpu.sync_copy(lab_hbm, lab_vmem)

        def pre(t, _):
            buf_ref[pl.ds(_MAIN + 256 + 16 * t, 16)] = jnp.full(
                (16,), _NEG, jnp.float32)
            return 0

        lax.fori_loop(0, (_BUF - _MAIN - 256) // 16, pre, 0)

        def round_body(r, loss):
            return loss + _process_row(prob_hbm, tail_hbm, 32 * r + wid,
                                       buf_ref, cm_ref, ids_ref, lab_vmem,
                                       sem_a, sem_b)

        loss = lax.fori_loop(0, 2, round_body, jnp.float32(0.0))

        stage_ref[...] = jnp.full((16,), loss, jnp.float32)
        pltpu.sync_copy(stage_ref, out_hbm.at[wid])

    return body(probas, tail, labels)


def _reduce_kernel(part_ref, out_ref):
    out_ref[0, 0] = jnp.sum(part_ref[:, 0:1]) * (1.0 / _B)


@jax.jit
def kernel(probas, labels):
    tail = jnp.pad(probas[:, _MAIN:], ((0, 0), (0, 96)),
                   constant_values=-np.inf)
    partials = _sc_loss(probas, tail, labels.astype(jnp.int32))
    out = pl.pallas_call(
        _reduce_kernel,
        out_specs=pl.BlockSpec(memory_space=pltpu.SMEM),
        out_shape=jax.ShapeDtypeStruct((1, 1), jnp.float32),
    )(partials)
    return out[0, 0]


# speedup vs baseline: 1.2161x; 1.0253x over previous
import functools

import jax
import jax.numpy as jnp
import numpy as np
from jax import lax
from jax.experimental import pallas as pl
from jax.experimental.pallas import tpu as pltpu
from jax.experimental.pallas import tpu_sc as plsc

_B = 64
_N = 100000
_CH = 128
_NCH = (_N + _CH - 1) // _CH
_PIECE = 16384
_NP = 7
_BUF = _NP * _PIECE
_NG = _BUF // (16 * _CH)
_NSEL = 12
_MAIN = 99840
_NEG = np.float32(-np.inf)


def _splat(x):
    return jnp.full((16,), x, jnp.int32)


def _piece_max(buf_ref, cm_ref, p):
    base = p * _PIECE

    def body(t, accs):
        off = base + t * 128
        return tuple(
            jnp.maximum(a, buf_ref[pl.ds(off + 16 * v, 16)])
            for v, a in enumerate(accs))

    accs = lax.fori_loop(
        0, 128, body,
        tuple(jnp.full((16,), _NEG, jnp.float32) for _ in range(8)))
    for v, a in enumerate(accs):
        cm_ref[pl.ds(p * 128 + 16 * v, 16)] = a


def _process_row(prob_hbm, tail_hbm, ri, buf_ref, cm_ref, ids_ref, lab_vmem,
                 sem_a, sem_b):
    iota = lax.iota(jnp.int32, 16)
    sems = [sem_a, sem_b]
    sizes = [_PIECE] * (_NP - 1) + [_MAIN - (_NP - 1) * _PIECE]

    def fire(p, sem):
        return pltpu.async_copy(
            prob_hbm.at[ri, pl.ds(p * _PIECE, sizes[p])],
            buf_ref.at[pl.ds(p * _PIECE, sizes[p])], sem)

    descs = [None] * _NP
    descs[0] = fire(0, sems[0])
    descs[1] = fire(1, sems[1])
    pltpu.sync_copy(tail_hbm.at[ri], buf_ref.at[pl.ds(_MAIN, 256)])
    for p in range(_NP):
        descs[p].wait()
        if p + 2 < _NP:
            descs[p + 2] = fire(p + 2, sems[p % 2])
        _piece_max(buf_ref, cm_ref, p)

    lane0 = iota == 0
    lab_s = jnp.max(plsc.load_gather(lab_vmem, [_splat(ri)]))
    idg = (lab_s // _PIECE) * 128 + (lab_s % 128)
    plsc.store_scatter(cm_ref, [_splat(idg)], jnp.full((16,), _NEG), mask=lane0)
    plsc.store_scatter(ids_ref, [_splat(_NSEL - 1)], _splat(idg), mask=lane0)

    def sel_body(t, _):
        def scan1(v, mvs):
            return tuple(
                jnp.maximum(m, cm_ref[pl.ds((v * 8 + u) * 16, 16)])
                for u, m in enumerate(mvs))

        mvs = lax.fori_loop(
            0, _NG // 8, scan1,
            tuple(jnp.full((16,), _NEG, jnp.float32) for _ in range(8)))
        mv = mvs[0]
        for m in mvs[1:]:
            mv = jnp.maximum(mv, m)
        mg = jnp.max(mv)

        def scan2(v, pvs):
            out = []
            for u, pv in enumerate(pvs):
                cv = cm_ref[pl.ds((v * 8 + u) * 16, 16)]
                out.append(jnp.minimum(
                    pv, jnp.where(cv == mg, iota + (v * 8 + u) * 16, 9999)))
            return tuple(out)

        pvs = lax.fori_loop(
            0, _NG // 8, scan2,
            tuple(jnp.full((16,), 9999, jnp.int32) for _ in range(8)))
        pv = pvs[0]
        for p2 in pvs[1:]:
            pv = jnp.minimum(pv, p2)
        pos = jnp.min(pv)
        plsc.store_scatter(cm_ref, [_splat(pos)], jnp.full((16,), _NEG), mask=lane0)
        plsc.store_scatter(ids_ref, [_splat(t)], _splat(pos), mask=lane0)
        return 0

    lax.fori_loop(0, _NSEL - 1, sel_body, 0)

    st = tuple(jnp.full((16,), _NEG, jnp.float32) for _ in range(11))

    def ins_body(t, st):
        k = t // 8
        j = t - k * 8
        idk = jnp.max(plsc.load_gather(ids_ref, [_splat(k)]))
        base = (idk // 128) * _PIECE + (idk % 128) + 2048 * j
        x = plsc.load_gather(buf_ref, [_splat(base) + iota * 128])
        out = []
        for s in st:
            hi = jnp.maximum(s, x)
            x = jnp.minimum(s, x)
            out.append(hi)
        return tuple(out)

    st = lax.fori_loop(0, _NSEL * 8, ins_body, st)

    def mrg_body(t, carry):
        st, sum11, v11 = carry
        mg = jnp.float32(-jnp.inf)
        for s in st:
            mg = jnp.maximum(mg, jnp.max(s))
        sum11 = sum11 + mg
        pv = jnp.full((16,), 9999, jnp.int32)
        for u, s in enumerate(st):
            pv = jnp.minimum(pv, jnp.where(s == mg, iota + u * 16, 9999))
        pos = jnp.min(pv)
        st = tuple(jnp.where(iota == pos - u * 16, _NEG, s)
                   for u, s in enumerate(st))
        return st, sum11, mg

    _, sum11, v11 = lax.fori_loop(
        0, 11, mrg_body, (st, jnp.float32(0.0), jnp.float32(0.0)))

    c = jnp.max(plsc.load_gather(buf_ref, [_splat(lab_s)]))
    sub = jnp.where(c >= v11, c, v11)
    return (sum11 - sub) * jnp.float32(0.1) - c


def _sc_loss(probas, tail, labels):
    mesh = plsc.VectorSubcoreMesh(core_axis_name="c", subcore_axis_name="s")

    @functools.partial(
        pl.kernel,
        out_type=jax.ShapeDtypeStruct((32, 16), jnp.float32),
        mesh=mesh,
        compiler_params=pltpu.CompilerParams(use_tc_tiling_on_sc=True,
                                             needs_layout_passes=False),
        scratch_types=[
            pltpu.VMEM((_BUF,), jnp.float32),
            pltpu.VMEM((16 * _NG,), jnp.float32),
            pltpu.VMEM((16,), jnp.int32),
            pltpu.VMEM((64,), jnp.int32),
            pltpu.VMEM((16,), jnp.float32),
            pltpu.SemaphoreType.DMA,
            pltpu.SemaphoreType.DMA,
        ],
    )
    def body(prob_hbm, tail_hbm, lab_hbm, out_hbm, buf_ref, cm_ref, ids_ref,
             lab_vmem, stage_ref, sem_a, sem_b):
        wid = lax.axis_index("s") * 2 + lax.axis_index("c")
        pltpu.sync_copy(lab_hbm, lab_vmem)

        def pre(t, _):
            buf_ref[pl.ds(_MAIN + 256 + 16 * t, 16)] = jnp.full(
                (16,), _NEG, jnp.float32)
            return 0

        lax.fori_loop(0, (_BUF - _MAIN - 256) // 16, pre, 0)

        loss = _process_row(prob_hbm, tail_hbm, wid, buf_ref, cm_ref,
                            ids_ref, lab_vmem, sem_a, sem_b)

        stage_ref[...] = jnp.full((16,), loss, jnp.float32)
        pltpu.sync_copy(stage_ref, out_hbm.at[wid])

    return body(probas, tail, labels)


def _reduce_kernel(part_ref, tc_ref, out_ref):
    out_ref[0, 0] = (jnp.sum(part_ref[:, 0:1]) + tc_ref[0]) * (1.0 / _B)


_K = 11
_W = 4096
_NB = (_N + _W - 1) // _W
_S = _W // 128


def _tc_half_kernel(prob_ref, lab_ref, out_ref, state_ref, cacc_ref):
    i = pl.program_id(0)
    nr = _B // 2

    @pl.when(i == 0)
    def _init():
        state_ref[...] = jnp.full((nr, _K * 128), -jnp.inf, jnp.float32)
        cacc_ref[...] = jnp.zeros((nr, 128), jnp.float32)

    st = [state_ref[:, j * 128:(j + 1) * 128] for j in range(_K)]
    cacc = cacc_ref[...]
    labs = lab_ref[...]
    base = i * _W
    lane = lax.broadcasted_iota(jnp.int32, (nr, 128), 1)
    for s in range(_S):
        x = prob_ref[:, s * 128:(s + 1) * 128]
        cols = base + s * 128 + lane
        valid = cols < _N
        cacc = cacc + jnp.where((cols == labs) & valid, x, 0.0)
        xm = jnp.where(valid, x, -jnp.inf)
        for j in range(_K):
            hi = jnp.maximum(st[j], xm)
            xm = jnp.minimum(st[j], xm)
            st[j] = hi
    for j in range(_K):
        state_ref[:, j * 128:(j + 1) * 128] = st[j]
    cacc_ref[...] = cacc

    @pl.when(i == _NB - 1)
    def _finish():
        a = state_ref[...]
        iota = lax.broadcasted_iota(jnp.int32, (nr, _K * 128), 1)
        sum11 = jnp.zeros((nr, 1), jnp.float32)
        m = jnp.zeros((nr, 1), jnp.float32)
        for _ in range(_K):
            m = jnp.max(a, axis=1, keepdims=True)
            sum11 = sum11 + m
            pos = jnp.min(jnp.where(a == m, iota, _K * 128),
                          axis=1, keepdims=True)
            a = jnp.where(iota == pos, -jnp.inf, a)
        v11 = m
        c = jnp.sum(cacc_ref[...], axis=1, keepdims=True)
        sub = jnp.where(c >= v11, c, v11)
        loss_rows = (sum11 - sub) * 0.1 - c
        out_ref[0] = jnp.sum(loss_rows)


def _tc_half(probas, lab2d):
    nr = _B // 2
    return pl.pallas_call(
        _tc_half_kernel,
        grid=(_NB,),
        in_specs=[
            pl.BlockSpec((nr, _W), lambda i: (1, i)),
            pl.BlockSpec((nr, 128), lambda i: (1, 0)),
        ],
        out_specs=pl.BlockSpec(memory_space=pltpu.SMEM),
        out_shape=jax.ShapeDtypeStruct((1,), jnp.float32),
        scratch_shapes=[
            pltpu.VMEM((nr, _K * 128), jnp.float32),
            pltpu.VMEM((nr, 128), jnp.float32),
        ],
    )(probas, lab2d)


@jax.jit
def kernel(probas, labels):
    tail = jnp.pad(probas[:, _MAIN:], ((0, 0), (0, 96)),
                   constant_values=-np.inf)
    labels32 = labels.astype(jnp.int32)
    partials = _sc_loss(probas, tail, labels32)
    lab2d = jnp.broadcast_to(labels32[:, None], (_B, 128))
    tcsum = _tc_half(probas, lab2d)
    out = pl.pallas_call(
        _reduce_kernel,
        in_specs=[
            pl.BlockSpec(memory_space=pltpu.VMEM),
            pl.BlockSpec(memory_space=pltpu.SMEM),
        ],
        out_specs=pl.BlockSpec(memory_space=pltpu.SMEM),
        out_shape=jax.ShapeDtypeStruct((1, 1), jnp.float32),
    )(partials, tcsum)
    return out[0, 0]


# speedup vs baseline: 1.2187x; 1.0021x over previous
import functools

import jax
import jax.numpy as jnp
import numpy as np
from jax import lax
from jax.experimental import pallas as pl
from jax.experimental.pallas import tpu as pltpu
from jax.experimental.pallas import tpu_sc as plsc

_B = 64
_N = 100000
_CH = 128
_NCH = (_N + _CH - 1) // _CH
_PIECE = 16384
_NP = 7
_BUF = _NP * _PIECE
_NG = _BUF // (16 * _CH)
_NSEL = 12
_MAIN = 99840
_NEG = np.float32(-np.inf)


def _splat(x):
    return jnp.full((16,), x, jnp.int32)


def _piece_max(buf_ref, cm_ref, p):
    base = p * _PIECE

    def body(t, accs):
        off = base + t * 256
        accs = tuple(
            jnp.maximum(a, buf_ref[pl.ds(off + 16 * v, 16)])
            for v, a in enumerate(accs))
        return tuple(
            jnp.maximum(a, buf_ref[pl.ds(off + 128 + 16 * v, 16)])
            for v, a in enumerate(accs))

    accs = lax.fori_loop(
        0, 64, body,
        tuple(jnp.full((16,), _NEG, jnp.float32) for _ in range(8)))
    for v, a in enumerate(accs):
        cm_ref[pl.ds(p * 128 + 16 * v, 16)] = a


def _process_row(prob_hbm, tail_hbm, ri, buf_ref, cm_ref, ids_ref, lab_vmem,
                 sem_a, sem_b):
    iota = lax.iota(jnp.int32, 16)
    sems = [sem_a, sem_b]
    sizes = [_PIECE] * (_NP - 1) + [_MAIN - (_NP - 1) * _PIECE]

    def fire(p, sem):
        return pltpu.async_copy(
            prob_hbm.at[ri, pl.ds(p * _PIECE, sizes[p])],
            buf_ref.at[pl.ds(p * _PIECE, sizes[p])], sem)

    descs = [None] * _NP
    descs[0] = fire(0, sems[0])
    descs[1] = fire(1, sems[1])
    pltpu.sync_copy(tail_hbm.at[ri], buf_ref.at[pl.ds(_MAIN, 256)])
    for p in range(_NP):
        descs[p].wait()
        if p + 2 < _NP:
            descs[p + 2] = fire(p + 2, sems[p % 2])
        _piece_max(buf_ref, cm_ref, p)

    lane0 = iota == 0
    lab_s = jnp.max(plsc.load_gather(lab_vmem, [_splat(ri)]))
    idg = (lab_s // _PIECE) * 128 + (lab_s % 128)
    plsc.store_scatter(cm_ref, [_splat(idg)], jnp.full((16,), _NEG), mask=lane0)
    plsc.store_scatter(ids_ref, [_splat(_NSEL - 1)], _splat(idg), mask=lane0)

    def sel_body(t, _):
        def scan1(v, mvs):
            return tuple(
                jnp.maximum(m, cm_ref[pl.ds((v * 8 + u) * 16, 16)])
                for u, m in enumerate(mvs))

        mvs = lax.fori_loop(
            0, _NG // 8, scan1,
            tuple(jnp.full((16,), _NEG, jnp.float32) for _ in range(8)))
        mv = mvs[0]
        for m in mvs[1:]:
            mv = jnp.maximum(mv, m)
        mg = jnp.max(mv)

        def scan2(v, pvs):
            out = []
            for u, pv in enumerate(pvs):
                cv = cm_ref[pl.ds((v * 8 + u) * 16, 16)]
                out.append(jnp.minimum(
                    pv, jnp.where(cv == mg, iota + (v * 8 + u) * 16, 9999)))
            return tuple(out)

        pvs = lax.fori_loop(
            0, _NG // 8, scan2,
            tuple(jnp.full((16,), 9999, jnp.int32) for _ in range(8)))
        pv = pvs[0]
        for p2 in pvs[1:]:
            pv = jnp.minimum(pv, p2)
        pos = jnp.min(pv)
        plsc.store_scatter(cm_ref, [_splat(pos)], jnp.full((16,), _NEG), mask=lane0)
        plsc.store_scatter(ids_ref, [_splat(t)], _splat(pos), mask=lane0)
        return 0

    lax.fori_loop(0, _NSEL - 1, sel_body, 0)

    st = tuple(jnp.full((16,), _NEG, jnp.float32) for _ in range(11))

    def ins_body(t, st):
        k = t // 8
        j = t - k * 8
        idk = jnp.max(plsc.load_gather(ids_ref, [_splat(k)]))
        base = (idk // 128) * _PIECE + (idk % 128) + 2048 * j
        x = plsc.load_gather(buf_ref, [_splat(base) + iota * 128])
        out = []
        for s in st:
            hi = jnp.maximum(s, x)
            x = jnp.minimum(s, x)
            out.append(hi)
        return tuple(out)

    st = lax.fori_loop(0, _NSEL * 8, ins_body, st)

    def mrg_body(t, carry):
        st, sum11, v11 = carry
        mg = jnp.float32(-jnp.inf)
        for s in st:
            mg = jnp.maximum(mg, jnp.max(s))
        sum11 = sum11 + mg
        pv = jnp.full((16,), 9999, jnp.int32)
        for u, s in enumerate(st):
            pv = jnp.minimum(pv, jnp.where(s == mg, iota + u * 16, 9999))
        pos = jnp.min(pv)
        st = tuple(jnp.where(iota == pos - u * 16, _NEG, s)
                   for u, s in enumerate(st))
        return st, sum11, mg

    _, sum11, v11 = lax.fori_loop(
        0, 11, mrg_body, (st, jnp.float32(0.0), jnp.float32(0.0)))

    c = jnp.max(plsc.load_gather(buf_ref, [_splat(lab_s)]))
    sub = jnp.where(c >= v11, c, v11)
    return (sum11 - sub) * jnp.float32(0.1) - c


def _sc_loss(probas, tail, labels):
    mesh = plsc.VectorSubcoreMesh(core_axis_name="c", subcore_axis_name="s")

    @functools.partial(
        pl.kernel,
        out_type=jax.ShapeDtypeStruct((32, 16), jnp.float32),
        mesh=mesh,
        compiler_params=pltpu.CompilerParams(use_tc_tiling_on_sc=True,
                                             needs_layout_passes=False),
        scratch_types=[
            pltpu.VMEM((_BUF,), jnp.float32),
            pltpu.VMEM((16 * _NG,), jnp.float32),
            pltpu.VMEM((16,), jnp.int32),
            pltpu.VMEM((64,), jnp.int32),
            pltpu.VMEM((16,), jnp.float32),
            pltpu.SemaphoreType.DMA,
            pltpu.SemaphoreType.DMA,
        ],
    )
    def body(prob_hbm, tail_hbm, lab_hbm, out_hbm, buf_ref, cm_ref, ids_ref,
             lab_vmem, stage_ref, sem_a, sem_b):
        wid = lax.axis_index("s") * 2 + lax.axis_index("c")
        pltpu.sync_copy(lab_hbm, lab_vmem)

        def pre(t, _):
            buf_ref[pl.ds(_MAIN + 256 + 16 * t, 16)] = jnp.full(
                (16,), _NEG, jnp.float32)
            return 0

        lax.fori_loop(0, (_BUF - _MAIN - 256) // 16, pre, 0)

        loss = _process_row(prob_hbm, tail_hbm, wid, buf_ref, cm_ref,
                            ids_ref, lab_vmem, sem_a, sem_b)

        stage_ref[...] = jnp.full((16,), loss, jnp.float32)
        pltpu.sync_copy(stage_ref, out_hbm.at[wid])

    return body(probas, tail, labels)


def _reduce_kernel(part_ref, tc_ref, out_ref):
    out_ref[0, 0] = (jnp.sum(part_ref[:, 0:1]) + tc_ref[0]) * (1.0 / _B)


_K = 11
_W = 4096
_NB = (_N + _W - 1) // _W
_S = _W // 128


def _tc_half_kernel(prob_ref, lab_ref, out_ref, state_ref, cacc_ref):
    i = pl.program_id(0)
    nr = _B // 2

    @pl.when(i == 0)
    def _init():
        state_ref[...] = jnp.full((nr, _K * 128), -jnp.inf, jnp.float32)
        cacc_ref[...] = jnp.zeros((nr, 128), jnp.float32)

    st = [state_ref[:, j * 128:(j + 1) * 128] for j in range(_K)]
    cacc = cacc_ref[...]
    labs = lab_ref[...]
    base = i * _W
    lane = lax.broadcasted_iota(jnp.int32, (nr, 128), 1)
    for s in range(_S):
        x = prob_ref[:, s * 128:(s + 1) * 128]
        cols = base + s * 128 + lane
        valid = cols < _N
        cacc = cacc + jnp.where((cols == labs) & valid, x, 0.0)
        xm = jnp.where(valid, x, -jnp.inf)
        for j in range(_K):
            hi = jnp.maximum(st[j], xm)
            xm = jnp.minimum(st[j], xm)
            st[j] = hi
    for j in range(_K):
        state_ref[:, j * 128:(j + 1) * 128] = st[j]
    cacc_ref[...] = cacc

    @pl.when(i == _NB - 1)
    def _finish():
        a = state_ref[...]
        iota = lax.broadcasted_iota(jnp.int32, (nr, _K * 128), 1)
        sum11 = jnp.zeros((nr, 1), jnp.float32)
        m = jnp.zeros((nr, 1), jnp.float32)
        for _ in range(_K):
            m = jnp.max(a, axis=1, keepdims=True)
            sum11 = sum11 + m
            pos = jnp.min(jnp.where(a == m, iota, _K * 128),
                          axis=1, keepdims=True)
            a = jnp.where(iota == pos, -jnp.inf, a)
        v11 = m
        c = jnp.sum(cacc_ref[...], axis=1, keepdims=True)
        sub = jnp.where(c >= v11, c, v11)
        loss_rows = (sum11 - sub) * 0.1 - c
        out_ref[0] = jnp.sum(loss_rows)


def _tc_half(probas, lab2d):
    nr = _B // 2
    return pl.pallas_call(
        _tc_half_kernel,
        grid=(_NB,),
        in_specs=[
            pl.BlockSpec((nr, _W), lambda i: (1, i)),
            pl.BlockSpec((nr, 128), lambda i: (1, 0)),
        ],
        out_specs=pl.BlockSpec(memory_space=pltpu.SMEM),
        out_shape=jax.ShapeDtypeStruct((1,), jnp.float32),
        scratch_shapes=[
            pltpu.VMEM((nr, _K * 128), jnp.float32),
            pltpu.VMEM((nr, 128), jnp.float32),
        ],
    )(probas, lab2d)


@jax.jit
def kernel(probas, labels):
    tail = jnp.pad(probas[:, _MAIN:], ((0, 0), (0, 96)),
                   constant_values=-np.inf)
    labels32 = labels.astype(jnp.int32)
    partials = _sc_loss(probas, tail, labels32)
    lab2d = jnp.broadcast_to(labels32[:, None], (_B, 128))
    tcsum = _tc_half(probas, lab2d)
    out = pl.pallas_call(
        _reduce_kernel,
        in_specs=[
            pl.BlockSpec(memory_space=pltpu.VMEM),
            pl.BlockSpec(memory_space=pltpu.SMEM),
        ],
        out_specs=pl.BlockSpec(memory_space=pltpu.SMEM),
        out_shape=jax.ShapeDtypeStruct((1, 1), jnp.float32),
    )(partials, tcsum)
    return out[0, 0]
